# baseline (speedup 1.0000x reference)
import functools

import jax
import jax.numpy as jnp
from jax import lax
from jax.experimental import pallas as pl
from jax.experimental.pallas import tpu as pltpu
from jax.experimental.pallas import tpu_sc as plsc

N = 10000
D = 128
E = 320000
NPAD = 10240
SLICE = NPAD // 16
SLICEA_STRIDE = 624
SLICEA = 640
CH = 80
NCHUNK = E // CH
NW = 32
BASE_CH = NCHUNK // NW
EXTRA = NCHUNK - BASE_CH * NW

G = 4
NG = NCHUNK // G
BASE_G = NG // NW
EXTRA_G = NG - BASE_G * NW

BLK = 400
GRID = N // BLK

_MESH = plsc.VectorSubcoreMesh(core_axis_name="c", subcore_axis_name="s")


def _sc_deg_body(eint_hbm, zeros_hbm, out0_hbm, out1_hbm, idx, ones_v, acc,
                 sem_l, sem_s, sem_z):
    c = lax.axis_index("c")
    s = lax.axis_index("s")
    wid = s * 2 + c
    for k in range(CH // 16):
        ones_v[pl.ds(16 * k, 16)] = jnp.ones((16,), jnp.float32)
    sl = pl.ds(pl.multiple_of(s * SLICE, SLICE), SLICE)

    ng = BASE_G + jnp.where(wid < EXTRA_G, 1, 0)

    def fire_load(i):
        goff = (wid + NW * i) * G
        pltpu.async_copy(eint_hbm.at[pl.ds(goff, G)], idx.at[lax.rem(i, 2)],
                         sem_l)

    def drain_load(i):
        goff = (wid + NW * i) * G
        pltpu.make_async_copy(eint_hbm.at[pl.ds(goff, G)],
                              idx.at[lax.rem(i, 2)], sem_l).wait()

    def fire_scat(i, j):
        pltpu.async_copy(ones_v, acc.at[idx.at[lax.rem(i, 2), j, 1]], sem_s,
                         add=True)

    def drain_scat(i, j):
        pltpu.make_async_copy(ones_v, acc.at[idx.at[lax.rem(i, 2), j, 1]],
                              sem_s).wait()

    pltpu.async_copy(zeros_hbm.at[sl], acc.at[sl], sem_z)
    fire_load(0)
    pltpu.make_async_copy(zeros_hbm.at[sl], acc.at[sl], sem_z).wait()
    plsc.subcore_barrier()

    def body(i, carry):
        @pl.when(i >= 1)
        def _():
            for j in range(G):
                drain_scat(i - 1, j)

        @pl.when(i + 1 < ng)
        def _():
            fire_load(i + 1)

        drain_load(i)
        for j in range(G):
            fire_scat(i, j)
        return carry

    lax.fori_loop(0, ng, body, 0)
    for j in range(G):
        drain_scat(ng - 1, j)
    plsc.subcore_barrier()

    @pl.when(c == 0)
    def _():
        pltpu.sync_copy(acc.at[sl], out0_hbm.at[sl])

    @pl.when(c == 1)
    def _():
        pltpu.sync_copy(acc.at[sl], out1_hbm.at[sl])


_sc_deg = functools.partial(
    pl.kernel,
    mesh=_MESH,
    out_type=[
        jax.ShapeDtypeStruct((NPAD,), jnp.float32),
        jax.ShapeDtypeStruct((NPAD,), jnp.float32),
    ],
    scratch_types=[
        pltpu.VMEM((2, G, 2, CH), jnp.int32),
        pltpu.VMEM((CH,), jnp.float32),
        pltpu.VMEM_SHARED((NPAD,), jnp.float32),
        pltpu.SemaphoreType.DMA,
        pltpu.SemaphoreType.DMA,
        pltpu.SemaphoreType.DMA,
    ],
)(_sc_deg_body)


def _sc_agg_body(h_hbm, eint_hbm, zeros_hbm, out0_hbm, out1_hbm,
                 idx, rows, acc, sem_g, sem_s, sem_i, sem_z):
    c = lax.axis_index("c")
    s = lax.axis_index("s")
    wid = s * 2 + c
    sl = pl.ds(pl.multiple_of(s * SLICEA_STRIDE, 8), SLICEA)

    nch = BASE_CH + jnp.where(wid < EXTRA, 1, 0)

    def fire_idx(i):
        pltpu.async_copy(eint_hbm.at[wid + NW * i], idx.at[lax.rem(i, 6)],
                         sem_i)

    def drain_idx(i):
        pltpu.make_async_copy(eint_hbm.at[wid + NW * i],
                              idx.at[lax.rem(i, 6)], sem_i).wait()

    def fire_gather(i):
        pltpu.async_copy(h_hbm.at[idx.at[lax.rem(i, 6), 0]],
                         rows.at[lax.rem(i, 4)], sem_g)

    def drain_gather(i):
        pltpu.make_async_copy(h_hbm.at[idx.at[lax.rem(i, 6), 0]],
                              rows.at[lax.rem(i, 4)], sem_g).wait()

    def fire_scatter(i):
        pltpu.async_copy(rows.at[lax.rem(i, 4)],
                         acc.at[idx.at[lax.rem(i, 6), 1]], sem_s, add=True)

    def drain_scatter(i):
        pltpu.make_async_copy(rows.at[lax.rem(i, 4)],
                              acc.at[idx.at[lax.rem(i, 6), 1]], sem_s).wait()

    pltpu.async_copy(zeros_hbm.at[sl], acc.at[sl], sem_z)
    for k in range(4):
        fire_idx(k)
    drain_idx(0)
    fire_gather(0)
    drain_idx(1)
    fire_gather(1)
    pltpu.make_async_copy(zeros_hbm.at[sl], acc.at[sl], sem_z).wait()
    plsc.subcore_barrier()

    def body(i, carry):
        @pl.when(i >= 2)
        def _():
            drain_scatter(i - 2)

        @pl.when(i + 2 < nch)
        def _():
            drain_idx(i + 2)
            fire_gather(i + 2)

        @pl.when(i + 4 < nch)
        def _():
            fire_idx(i + 4)

        drain_gather(i)
        fire_scatter(i)
        return carry

    lax.fori_loop(0, nch, body, 0)
    drain_scatter(nch - 2)
    drain_scatter(nch - 1)
    plsc.subcore_barrier()

    @pl.when(c == 0)
    def _():
        pltpu.sync_copy(acc.at[sl], out0_hbm.at[sl])

    @pl.when(c == 1)
    def _():
        pltpu.sync_copy(acc.at[sl], out1_hbm.at[sl])


_sc_agg = functools.partial(
    pl.kernel,
    mesh=_MESH,
    out_type=[
        jax.ShapeDtypeStruct((N, D), jnp.float32),
        jax.ShapeDtypeStruct((N, D), jnp.float32),
    ],
    scratch_types=[
        pltpu.VMEM((6, 2, CH), jnp.int32),
        pltpu.VMEM((4, CH, D), jnp.float32),
        pltpu.VMEM_SHARED((N, D), jnp.float32),
        pltpu.SemaphoreType.DMA,
        pltpu.SemaphoreType.DMA,
        pltpu.SemaphoreType.DMA,
        pltpu.SemaphoreType.DMA,
    ],
)(_sc_agg_body)


def _tc_mm_body(x_ref, w_ref, u_ref):
    u_ref[...] = jnp.dot(x_ref[...], w_ref[...],
                         preferred_element_type=jnp.float32)


_tc_mm = pl.pallas_call(
    _tc_mm_body,
    grid=(GRID,),
    in_specs=[
        pl.BlockSpec((BLK, D), lambda i: (i, 0)),
        pl.BlockSpec((D, D), lambda i: (0, 0)),
    ],
    out_specs=pl.BlockSpec((BLK, D), lambda i: (i, 0)),
    out_shape=jax.ShapeDtypeStruct((N, D), jnp.float32),
)


def _tc_scale_body(d0_ref, d1_ref, u_ref, dinv_ref, h_ref):
    deg = d0_ref[...] + d1_ref[...] + 1.0
    dinv = lax.rsqrt(deg)
    dinv_ref[...] = dinv
    h_ref[...] = u_ref[...] * dinv


_tc_scale = pl.pallas_call(
    _tc_scale_body,
    grid=(GRID,),
    in_specs=[
        pl.BlockSpec((BLK, 1), lambda i: (i, 0)),
        pl.BlockSpec((BLK, 1), lambda i: (i, 0)),
        pl.BlockSpec((BLK, D), lambda i: (i, 0)),
    ],
    out_specs=[
        pl.BlockSpec((BLK, 1), lambda i: (i, 0)),
        pl.BlockSpec((BLK, D), lambda i: (i, 0)),
    ],
    out_shape=[
        jax.ShapeDtypeStruct((N, 1), jnp.float32),
        jax.ShapeDtypeStruct((N, D), jnp.float32),
    ],
)


def _tc_layer_body(sa_ref, sb_ref, hp_ref, dinv_ref, b_ref, w_ref,
                   x_ref, hn_ref):
    agg = sa_ref[...] + sb_ref[...] + hp_ref[...]
    xl = jnp.maximum(dinv_ref[...] * agg + b_ref[...][None, :], 0.0)
    x_ref[...] = xl
    hn = jnp.dot(xl, w_ref[...], preferred_element_type=jnp.float32)
    hn_ref[...] = hn * dinv_ref[...]


_tc_layer = pl.pallas_call(
    _tc_layer_body,
    grid=(GRID,),
    in_specs=[
        pl.BlockSpec((BLK, D), lambda i: (i, 0)),
        pl.BlockSpec((BLK, D), lambda i: (i, 0)),
        pl.BlockSpec((BLK, D), lambda i: (i, 0)),
        pl.BlockSpec((BLK, 1), lambda i: (i, 0)),
        pl.BlockSpec((D,), lambda i: (0,)),
        pl.BlockSpec((D, D), lambda i: (0, 0)),
    ],
    out_specs=[
        pl.BlockSpec((BLK, D), lambda i: (i, 0)),
        pl.BlockSpec((BLK, D), lambda i: (i, 0)),
    ],
    out_shape=[
        jax.ShapeDtypeStruct((N, D), jnp.float32),
        jax.ShapeDtypeStruct((N, D), jnp.float32),
    ],
)


def _tc_final_body(tgt_ref, sa_ref, sb_ref, hp_ref, dinv_ref, b_ref,
                   x1_ref, x2_ref, wfc_ref, bfc_ref,
                   emb_ref, gmax_ref, out_ref, row_acc):
    i = pl.program_id(0)
    agg = sa_ref[...] + sb_ref[...] + hp_ref[...]
    x3 = dinv_ref[...] * agg + b_ref[...][None, :]
    cat = jnp.concatenate([x1_ref[...], x2_ref[...], x3], axis=1)
    emb_ref[...] = cat
    bm = jnp.max(cat, axis=0, keepdims=True)
    ids = i * BLK + lax.broadcasted_iota(jnp.int32, (BLK, 3 * D), 0)
    contrib = jnp.sum(jnp.where(ids == tgt_ref[0], cat, 0.0), axis=0,
                      keepdims=True)

    @pl.when(i == 0)
    def _():
        gmax_ref[...] = bm
        row_acc[...] = contrib

    @pl.when(i > 0)
    def _():
        gmax_ref[...] = jnp.maximum(gmax_ref[...], bm)
        row_acc[...] = row_acc[...] + contrib

    @pl.when(i == GRID - 1)
    def _():
        out_ref[...] = (jnp.dot(row_acc[...], wfc_ref[...],
                                preferred_element_type=jnp.float32)
                        + bfc_ref[...][None, :])


_tc_final = pl.pallas_call(
    _tc_final_body,
    grid=(GRID,),
    in_specs=[
        pl.BlockSpec(memory_space=pltpu.SMEM),
        pl.BlockSpec((BLK, D), lambda i: (i, 0)),
        pl.BlockSpec((BLK, D), lambda i: (i, 0)),
        pl.BlockSpec((BLK, D), lambda i: (i, 0)),
        pl.BlockSpec((BLK, 1), lambda i: (i, 0)),
        pl.BlockSpec((D,), lambda i: (0,)),
        pl.BlockSpec((BLK, D), lambda i: (i, 0)),
        pl.BlockSpec((BLK, D), lambda i: (i, 0)),
        pl.BlockSpec((3 * D, 10), lambda i: (0, 0)),
        pl.BlockSpec((10,), lambda i: (0,)),
    ],
    out_specs=[
        pl.BlockSpec((BLK, 3 * D), lambda i: (i, 0)),
        pl.BlockSpec((1, 3 * D), lambda i: (0, 0)),
        pl.BlockSpec((1, 10), lambda i: (0, 0)),
    ],
    out_shape=[
        jax.ShapeDtypeStruct((N, 3 * D), jnp.float32),
        jax.ShapeDtypeStruct((1, 3 * D), jnp.float32),
        jax.ShapeDtypeStruct((1, 10), jnp.float32),
    ],
    scratch_shapes=[pltpu.VMEM((1, 3 * D), jnp.float32)],
)


def kernel(x, edge_index, batch, target_node, W1, b1, W2, b2, W3, b3, Wfc, bfc):
    eint = edge_index.reshape(2, NCHUNK, CH).transpose(1, 0, 2)
    zrow = jnp.zeros((NPAD, D), jnp.float32)
    zdeg = jnp.zeros((NPAD,), jnp.float32)

    d0, d1 = _sc_deg(eint, zdeg)
    u1 = _tc_mm(x, W1)
    dinv, h1p = _tc_scale(d0.reshape(NPAD, 1), d1.reshape(NPAD, 1), u1)
    s1a, s1b = _sc_agg(h1p, eint, zrow)
    x1, h2p = _tc_layer(s1a, s1b, h1p, dinv, b1, W2)
    s2a, s2b = _sc_agg(h2p, eint, zrow)
    x2, h3p = _tc_layer(s2a, s2b, h2p, dinv, b2, W3)
    s3a, s3b = _sc_agg(h3p, eint, zrow)
    tgt = jnp.asarray(target_node, jnp.int32).reshape(1)
    emb, gmax, out = _tc_final(tgt, s3a, s3b, h3p, dinv, b3, x1, x2, Wfc, bfc)
    return emb, gmax, out


# speedup vs baseline: 1.0449x; 1.0449x over previous
import functools

import jax
import jax.numpy as jnp
from jax import lax
from jax.experimental import pallas as pl
from jax.experimental.pallas import tpu as pltpu
from jax.experimental.pallas import tpu_sc as plsc

N = 10000
D = 128
E = 320000
NPAD = 10240
SLICE = NPAD // 16
SLICEA_STRIDE = 624
SLICEA = 640
CH = 128
NCHUNK = E // CH
NW = 32
BASE_CH = NCHUNK // NW
EXTRA = NCHUNK - BASE_CH * NW

G = 4
NG = NCHUNK // G
BASE_G = NG // NW
EXTRA_G = NG - BASE_G * NW

BLK = 400
GRID = N // BLK

_MESH = plsc.VectorSubcoreMesh(core_axis_name="c", subcore_axis_name="s")


def _sc_deg_body(eint_hbm, zeros_hbm, out0_hbm, out1_hbm, idx, ones_v, acc,
                 sem_l, sem_s, sem_z):
    c = lax.axis_index("c")
    s = lax.axis_index("s")
    wid = s * 2 + c
    for k in range(CH // 16):
        ones_v[pl.ds(16 * k, 16)] = jnp.ones((16,), jnp.float32)
    sl = pl.ds(pl.multiple_of(s * SLICE, SLICE), SLICE)

    ng = BASE_G + jnp.where(wid < EXTRA_G, 1, 0)

    def fire_load(i):
        goff = (wid + NW * i) * G
        pltpu.async_copy(eint_hbm.at[pl.ds(goff, G)], idx.at[lax.rem(i, 2)],
                         sem_l)

    def drain_load(i):
        goff = (wid + NW * i) * G
        pltpu.make_async_copy(eint_hbm.at[pl.ds(goff, G)],
                              idx.at[lax.rem(i, 2)], sem_l).wait()

    def fire_scat(i, j):
        pltpu.async_copy(ones_v, acc.at[idx.at[lax.rem(i, 2), j, 1]], sem_s,
                         add=True)

    def drain_scat(i, j):
        pltpu.make_async_copy(ones_v, acc.at[idx.at[lax.rem(i, 2), j, 1]],
                              sem_s).wait()

    pltpu.async_copy(zeros_hbm.at[sl], acc.at[sl], sem_z)
    fire_load(0)
    pltpu.make_async_copy(zeros_hbm.at[sl], acc.at[sl], sem_z).wait()
    plsc.subcore_barrier()

    def body(i, carry):
        @pl.when(i >= 1)
        def _():
            for j in range(G):
                drain_scat(i - 1, j)

        @pl.when(i + 1 < ng)
        def _():
            fire_load(i + 1)

        drain_load(i)
        for j in range(G):
            fire_scat(i, j)
        return carry

    lax.fori_loop(0, ng, body, 0)
    for j in range(G):
        drain_scat(ng - 1, j)
    plsc.subcore_barrier()

    @pl.when(c == 0)
    def _():
        pltpu.sync_copy(acc.at[sl], out0_hbm.at[sl])

    @pl.when(c == 1)
    def _():
        pltpu.sync_copy(acc.at[sl], out1_hbm.at[sl])


_sc_deg = functools.partial(
    pl.kernel,
    mesh=_MESH,
    out_type=[
        jax.ShapeDtypeStruct((NPAD,), jnp.float32),
        jax.ShapeDtypeStruct((NPAD,), jnp.float32),
    ],
    scratch_types=[
        pltpu.VMEM((2, G, 2, CH), jnp.int32),
        pltpu.VMEM((CH,), jnp.float32),
        pltpu.VMEM_SHARED((NPAD,), jnp.float32),
        pltpu.SemaphoreType.DMA,
        pltpu.SemaphoreType.DMA,
        pltpu.SemaphoreType.DMA,
    ],
)(_sc_deg_body)


def _sc_agg_body(h_hbm, eint_hbm, zeros_hbm, out0_hbm, out1_hbm,
                 idx, rows, acc, sem_g, sem_s, sem_i, sem_z):
    c = lax.axis_index("c")
    s = lax.axis_index("s")
    wid = s * 2 + c
    sl = pl.ds(pl.multiple_of(s * SLICEA_STRIDE, 8), SLICEA)

    nch = BASE_CH + jnp.where(wid < EXTRA, 1, 0)

    def fire_idx(i):
        pltpu.async_copy(eint_hbm.at[wid + NW * i], idx.at[lax.rem(i, 6)],
                         sem_i)

    def drain_idx(i):
        pltpu.make_async_copy(eint_hbm.at[wid + NW * i],
                              idx.at[lax.rem(i, 6)], sem_i).wait()

    def fire_gather(i):
        pltpu.async_copy(h_hbm.at[idx.at[lax.rem(i, 6), 0]],
                         rows.at[lax.rem(i, 3)], sem_g)

    def drain_gather(i):
        pltpu.make_async_copy(h_hbm.at[idx.at[lax.rem(i, 6), 0]],
                              rows.at[lax.rem(i, 3)], sem_g).wait()

    def fire_scatter(i):
        pltpu.async_copy(rows.at[lax.rem(i, 3)],
                         acc.at[idx.at[lax.rem(i, 6), 1]], sem_s, add=True)

    def drain_scatter(i):
        pltpu.make_async_copy(rows.at[lax.rem(i, 3)],
                              acc.at[idx.at[lax.rem(i, 6), 1]], sem_s).wait()

    pltpu.async_copy(zeros_hbm.at[sl], acc.at[sl], sem_z)
    for k in range(4):
        fire_idx(k)
    drain_idx(0)
    fire_gather(0)
    drain_idx(1)
    fire_gather(1)
    pltpu.make_async_copy(zeros_hbm.at[sl], acc.at[sl], sem_z).wait()
    plsc.subcore_barrier()

    def body(i, carry):
        @pl.when(i >= 1)
        def _():
            drain_scatter(i - 1)

        @pl.when(i + 2 < nch)
        def _():
            drain_idx(i + 2)
            fire_gather(i + 2)

        @pl.when(i + 4 < nch)
        def _():
            fire_idx(i + 4)

        drain_gather(i)
        fire_scatter(i)
        return carry

    lax.fori_loop(0, nch, body, 0)
    drain_scatter(nch - 1)
    plsc.subcore_barrier()

    @pl.when(c == 0)
    def _():
        pltpu.sync_copy(acc.at[sl], out0_hbm.at[sl])

    @pl.when(c == 1)
    def _():
        pltpu.sync_copy(acc.at[sl], out1_hbm.at[sl])


_sc_agg = functools.partial(
    pl.kernel,
    mesh=_MESH,
    out_type=[
        jax.ShapeDtypeStruct((N, D), jnp.float32),
        jax.ShapeDtypeStruct((N, D), jnp.float32),
    ],
    scratch_types=[
        pltpu.VMEM((6, 2, CH), jnp.int32),
        pltpu.VMEM((3, CH, D), jnp.float32),
        pltpu.VMEM_SHARED((N, D), jnp.float32),
        pltpu.SemaphoreType.DMA,
        pltpu.SemaphoreType.DMA,
        pltpu.SemaphoreType.DMA,
        pltpu.SemaphoreType.DMA,
    ],
)(_sc_agg_body)


def _tc_mm_body(x_ref, w_ref, u_ref):
    u_ref[...] = jnp.dot(x_ref[...], w_ref[...],
                         preferred_element_type=jnp.float32)


_tc_mm = pl.pallas_call(
    _tc_mm_body,
    grid=(GRID,),
    in_specs=[
        pl.BlockSpec((BLK, D), lambda i: (i, 0)),
        pl.BlockSpec((D, D), lambda i: (0, 0)),
    ],
    out_specs=pl.BlockSpec((BLK, D), lambda i: (i, 0)),
    out_shape=jax.ShapeDtypeStruct((N, D), jnp.float32),
)


def _tc_scale_body(d0_ref, d1_ref, u_ref, dinv_ref, h_ref):
    deg = d0_ref[...] + d1_ref[...] + 1.0
    dinv = lax.rsqrt(deg)
    dinv_ref[...] = dinv
    h_ref[...] = u_ref[...] * dinv


_tc_scale = pl.pallas_call(
    _tc_scale_body,
    grid=(GRID,),
    in_specs=[
        pl.BlockSpec((BLK, 1), lambda i: (i, 0)),
        pl.BlockSpec((BLK, 1), lambda i: (i, 0)),
        pl.BlockSpec((BLK, D), lambda i: (i, 0)),
    ],
    out_specs=[
        pl.BlockSpec((BLK, 1), lambda i: (i, 0)),
        pl.BlockSpec((BLK, D), lambda i: (i, 0)),
    ],
    out_shape=[
        jax.ShapeDtypeStruct((N, 1), jnp.float32),
        jax.ShapeDtypeStruct((N, D), jnp.float32),
    ],
)


def _tc_layer_body(sa_ref, sb_ref, hp_ref, dinv_ref, b_ref, w_ref,
                   x_ref, hn_ref):
    agg = sa_ref[...] + sb_ref[...] + hp_ref[...]
    xl = jnp.maximum(dinv_ref[...] * agg + b_ref[...][None, :], 0.0)
    x_ref[...] = xl
    hn = jnp.dot(xl, w_ref[...], preferred_element_type=jnp.float32)
    hn_ref[...] = hn * dinv_ref[...]


_tc_layer = pl.pallas_call(
    _tc_layer_body,
    grid=(GRID,),
    in_specs=[
        pl.BlockSpec((BLK, D), lambda i: (i, 0)),
        pl.BlockSpec((BLK, D), lambda i: (i, 0)),
        pl.BlockSpec((BLK, D), lambda i: (i, 0)),
        pl.BlockSpec((BLK, 1), lambda i: (i, 0)),
        pl.BlockSpec((D,), lambda i: (0,)),
        pl.BlockSpec((D, D), lambda i: (0, 0)),
    ],
    out_specs=[
        pl.BlockSpec((BLK, D), lambda i: (i, 0)),
        pl.BlockSpec((BLK, D), lambda i: (i, 0)),
    ],
    out_shape=[
        jax.ShapeDtypeStruct((N, D), jnp.float32),
        jax.ShapeDtypeStruct((N, D), jnp.float32),
    ],
)


def _tc_final_body(tgt_ref, sa_ref, sb_ref, hp_ref, dinv_ref, b_ref,
                   x1_ref, x2_ref, wfc_ref, bfc_ref,
                   emb_ref, gmax_ref, out_ref, row_acc):
    i = pl.program_id(0)
    agg = sa_ref[...] + sb_ref[...] + hp_ref[...]
    x3 = dinv_ref[...] * agg + b_ref[...][None, :]
    cat = jnp.concatenate([x1_ref[...], x2_ref[...], x3], axis=1)
    emb_ref[...] = cat
    bm = jnp.max(cat, axis=0, keepdims=True)
    ids = i * BLK + lax.broadcasted_iota(jnp.int32, (BLK, 3 * D), 0)
    contrib = jnp.sum(jnp.where(ids == tgt_ref[0], cat, 0.0), axis=0,
                      keepdims=True)

    @pl.when(i == 0)
    def _():
        gmax_ref[...] = bm
        row_acc[...] = contrib

    @pl.when(i > 0)
    def _():
        gmax_ref[...] = jnp.maximum(gmax_ref[...], bm)
        row_acc[...] = row_acc[...] + contrib

    @pl.when(i == GRID - 1)
    def _():
        out_ref[...] = (jnp.dot(row_acc[...], wfc_ref[...],
                                preferred_element_type=jnp.float32)
                        + bfc_ref[...][None, :])


_tc_final = pl.pallas_call(
    _tc_final_body,
    grid=(GRID,),
    in_specs=[
        pl.BlockSpec(memory_space=pltpu.SMEM),
        pl.BlockSpec((BLK, D), lambda i: (i, 0)),
        pl.BlockSpec((BLK, D), lambda i: (i, 0)),
        pl.BlockSpec((BLK, D), lambda i: (i, 0)),
        pl.BlockSpec((BLK, 1), lambda i: (i, 0)),
        pl.BlockSpec((D,), lambda i: (0,)),
        pl.BlockSpec((BLK, D), lambda i: (i, 0)),
        pl.BlockSpec((BLK, D), lambda i: (i, 0)),
        pl.BlockSpec((3 * D, 10), lambda i: (0, 0)),
        pl.BlockSpec((10,), lambda i: (0,)),
    ],
    out_specs=[
        pl.BlockSpec((BLK, 3 * D), lambda i: (i, 0)),
        pl.BlockSpec((1, 3 * D), lambda i: (0, 0)),
        pl.BlockSpec((1, 10), lambda i: (0, 0)),
    ],
    out_shape=[
        jax.ShapeDtypeStruct((N, 3 * D), jnp.float32),
        jax.ShapeDtypeStruct((1, 3 * D), jnp.float32),
        jax.ShapeDtypeStruct((1, 10), jnp.float32),
    ],
    scratch_shapes=[pltpu.VMEM((1, 3 * D), jnp.float32)],
)


def kernel(x, edge_index, batch, target_node, W1, b1, W2, b2, W3, b3, Wfc, bfc):
    eint = edge_index.reshape(2, NCHUNK, CH).transpose(1, 0, 2)
    zrow = jnp.zeros((NPAD, D), jnp.float32)
    zdeg = jnp.zeros((NPAD,), jnp.float32)

    d0, d1 = _sc_deg(eint, zdeg)
    u1 = _tc_mm(x, W1)
    dinv, h1p = _tc_scale(d0.reshape(NPAD, 1), d1.reshape(NPAD, 1), u1)
    s1a, s1b = _sc_agg(h1p, eint, zrow)
    x1, h2p = _tc_layer(s1a, s1b, h1p, dinv, b1, W2)
    s2a, s2b = _sc_agg(h2p, eint, zrow)
    x2, h3p = _tc_layer(s2a, s2b, h2p, dinv, b2, W3)
    s3a, s3b = _sc_agg(h3p, eint, zrow)
    tgt = jnp.asarray(target_node, jnp.int32).reshape(1)
    emb, gmax, out = _tc_final(tgt, s3a, s3b, h3p, dinv, b3, x1, x2, Wfc, bfc)
    return emb, gmax, out
